# baseline (speedup 1.0000x reference)
import functools

import jax
import jax.numpy as jnp
from jax import lax
from jax.experimental import pallas as pl
from jax.experimental.pallas import tpu as pltpu
from jax.experimental.pallas import tpu_sc as plsc

_MARGIN = 1.0
_B = 16384
_D = 128
_L = 16


def _sc_sumsq(h_i, l_i, t_i, ch_i, ct_i, obj, rel):
    info = plsc.get_sparse_core_info()
    nc, ns = info.num_cores, info.num_subcores
    nw = nc * ns
    bpw = _B // nw
    chunk = 128
    nch = bpw // chunk

    mesh = plsc.VectorSubcoreMesh(core_axis_name="c", subcore_axis_name="s")

    @functools.partial(
        pl.kernel,
        out_type=(
            jax.ShapeDtypeStruct((_B,), jnp.float32),
            jax.ShapeDtypeStruct((_B,), jnp.float32),
        ),
        mesh=mesh,
        compiler_params=pltpu.CompilerParams(needs_layout_passes=False),
        scratch_types=[
            pltpu.VMEM((chunk,), jnp.int32),
            pltpu.VMEM((chunk,), jnp.int32),
            pltpu.VMEM((chunk,), jnp.int32),
            pltpu.VMEM((chunk,), jnp.int32),
            pltpu.VMEM((chunk,), jnp.int32),
            pltpu.VMEM((chunk, _D), jnp.float32),
            pltpu.VMEM((chunk, _D), jnp.float32),
            pltpu.VMEM((chunk, _D), jnp.float32),
            pltpu.VMEM((chunk, _D), jnp.float32),
            pltpu.VMEM((chunk, _D), jnp.float32),
            pltpu.VMEM((bpw,), jnp.float32),
            pltpu.VMEM((bpw,), jnp.float32),
            pltpu.SemaphoreType.DMA,
        ],
    )
    def sck(h_hbm, l_hbm, t_hbm, ch_hbm, ct_hbm, obj_hbm, rel_hbm,
            oe_hbm, oc_hbm,
            hi_v, li_v, ti_v, chi_v, cti_v,
            hr, lr, tr, cr, dr, eo, co, sem):
        wid = lax.axis_index("s") * nc + lax.axis_index("c")
        base = wid * bpw
        for g in range(nch):
            off = g * chunk
            pltpu.sync_copy(h_hbm.at[pl.ds(base + off, chunk)], hi_v)
            pltpu.sync_copy(l_hbm.at[pl.ds(base + off, chunk)], li_v)
            pltpu.sync_copy(t_hbm.at[pl.ds(base + off, chunk)], ti_v)
            pltpu.sync_copy(ch_hbm.at[pl.ds(base + off, chunk)], chi_v)
            pltpu.sync_copy(ct_hbm.at[pl.ds(base + off, chunk)], cti_v)
            cps = [
                pltpu.async_copy(obj_hbm.at[hi_v], hr, sem),
                pltpu.async_copy(rel_hbm.at[li_v], lr, sem),
                pltpu.async_copy(obj_hbm.at[ti_v], tr, sem),
                pltpu.async_copy(obj_hbm.at[chi_v], cr, sem),
                pltpu.async_copy(obj_hbm.at[cti_v], dr, sem),
            ]
            for cp in cps:
                cp.wait()

            ngrp = chunk // _L
            iota = lax.iota(jnp.int32, _L)
            row_ids = [iota + g * _L for g in range(ngrp)]

            def body(d, accs):
                col = jnp.full((_L,), d, jnp.int32)
                out = []
                for g in range(ngrp):
                    rows = row_ids[g]
                    lv = plsc.load_gather(lr, [rows, col])
                    hv = plsc.load_gather(hr, [rows, col])
                    tv = plsc.load_gather(tr, [rows, col])
                    cv = plsc.load_gather(cr, [rows, col])
                    dv = plsc.load_gather(dr, [rows, col])
                    de = hv + lv - tv
                    dc = cv + lv - dv
                    out.append(accs[2 * g] + de * de)
                    out.append(accs[2 * g + 1] + dc * dc)
                return tuple(out)

            accs = lax.fori_loop(
                0, _D, body,
                tuple(jnp.zeros((_L,), jnp.float32) for _ in range(2 * ngrp)))
            for g in range(ngrp):
                eo[pl.ds(off + g * _L, _L)] = accs[2 * g]
                co[pl.ds(off + g * _L, _L)] = accs[2 * g + 1]
        pltpu.sync_copy(eo, oe_hbm.at[pl.ds(base, bpw)])
        pltpu.sync_copy(co, oc_hbm.at[pl.ds(base, bpw)])

    return sck(h_i, l_i, t_i, ch_i, ct_i, obj, rel)


def _tc_finish(sum_e, sum_c):
    rows = _B // 128

    def body(se_ref, sc_ref, loss_ref, e_ref, ce_ref):
        e = jnp.sqrt(se_ref[...])
        ce = jnp.sqrt(sc_ref[...])
        e_ref[...] = e
        ce_ref[...] = ce
        loss_ref[...] = jnp.maximum(0.0, e - ce + _MARGIN)

    loss, e, ce = pl.pallas_call(
        body,
        out_shape=(
            jax.ShapeDtypeStruct((rows, 128), jnp.float32),
            jax.ShapeDtypeStruct((rows, 128), jnp.float32),
            jax.ShapeDtypeStruct((rows, 128), jnp.float32),
        ),
    )(sum_e.reshape(rows, 128), sum_c.reshape(rows, 128))
    return loss.reshape(_B), e.reshape(_B), ce.reshape(_B)


@jax.jit
def kernel(correct, corrupted, object_embedding, relationship_embedding):
    h_i = correct[:, 0]
    l_i = correct[:, 1]
    t_i = correct[:, 2]
    ch_i = corrupted[:, 0]
    ct_i = corrupted[:, 2]
    sum_e, sum_c = _sc_sumsq(h_i, l_i, t_i, ch_i, ct_i,
                             object_embedding, relationship_embedding)
    return _tc_finish(sum_e, sum_c)


# speedup vs baseline: 4.2455x; 4.2455x over previous
import functools

import jax
import jax.numpy as jnp
from jax import lax
from jax.experimental import pallas as pl
from jax.experimental.pallas import tpu as pltpu
from jax.experimental.pallas import tpu_sc as plsc

_MARGIN = 1.0
_B = 16384
_D = 128
_L = 16


def _sc_sumsq(h_i, l_i, t_i, ch_i, ct_i, obj, rel):
    info = plsc.get_sparse_core_info()
    nc, ns = info.num_cores, info.num_subcores
    nw = nc * ns
    bpw = _B // nw
    chunk = 64
    nch = bpw // chunk

    mesh = plsc.VectorSubcoreMesh(core_axis_name="c", subcore_axis_name="s")

    row_buf = pltpu.VMEM((chunk, _D), jnp.float32)

    @functools.partial(
        pl.kernel,
        out_type=(
            jax.ShapeDtypeStruct((_B,), jnp.float32),
            jax.ShapeDtypeStruct((_B,), jnp.float32),
        ),
        mesh=mesh,
        compiler_params=pltpu.CompilerParams(needs_layout_passes=False),
        scratch_types=[
            pltpu.VMEM((bpw,), jnp.int32),
            pltpu.VMEM((bpw,), jnp.int32),
            pltpu.VMEM((bpw,), jnp.int32),
            pltpu.VMEM((bpw,), jnp.int32),
            pltpu.VMEM((bpw,), jnp.int32),
            row_buf, row_buf, row_buf, row_buf, row_buf,
            row_buf, row_buf, row_buf, row_buf, row_buf,
            pltpu.VMEM((bpw,), jnp.float32),
            pltpu.VMEM((bpw,), jnp.float32),
            pltpu.VMEM((chunk * 17,), jnp.float32),
            pltpu.VMEM((chunk * 17,), jnp.float32),
            pltpu.SemaphoreType.DMA,
            pltpu.SemaphoreType.DMA,
        ],
    )
    def sck(h_hbm, l_hbm, t_hbm, ch_hbm, ct_hbm, obj_hbm, rel_hbm,
            oe_hbm, oc_hbm,
            hi_v, li_v, ti_v, chi_v, cti_v,
            hr0, lr0, tr0, cr0, dr0,
            hr1, lr1, tr1, cr1, dr1,
            eo, co, se_s, sc_s, sem0, sem1):
        wid = lax.axis_index("s") * nc + lax.axis_index("c")
        base = wid * bpw

        pltpu.sync_copy(h_hbm.at[pl.ds(base, bpw)], hi_v)
        pltpu.sync_copy(l_hbm.at[pl.ds(base, bpw)], li_v)
        pltpu.sync_copy(t_hbm.at[pl.ds(base, bpw)], ti_v)
        pltpu.sync_copy(ch_hbm.at[pl.ds(base, bpw)], chi_v)
        pltpu.sync_copy(ct_hbm.at[pl.ds(base, bpw)], cti_v)

        bufs = (
            (hr0, lr0, tr0, cr0, dr0, sem0),
            (hr1, lr1, tr1, cr1, dr1, sem1),
        )

        def fire(g, bs):
            off = g * chunk
            hrb, lrb, trb, crb, drb, sem = bs
            return [
                pltpu.async_copy(obj_hbm.at[hi_v.at[pl.ds(off, chunk)]], hrb, sem),
                pltpu.async_copy(rel_hbm.at[li_v.at[pl.ds(off, chunk)]], lrb, sem),
                pltpu.async_copy(obj_hbm.at[ti_v.at[pl.ds(off, chunk)]], trb, sem),
                pltpu.async_copy(obj_hbm.at[chi_v.at[pl.ds(off, chunk)]], crb, sem),
                pltpu.async_copy(obj_hbm.at[cti_v.at[pl.ds(off, chunk)]], drb, sem),
            ]

        iota17 = lax.iota(jnp.int32, _L) * 17

        def compute(g, bs):
            off = g * chunk
            hrb, lrb, trb, crb, drb, _ = bs

            def row_body(i, carry):
                acc_e = jnp.zeros((_L,), jnp.float32)
                acc_c = jnp.zeros((_L,), jnp.float32)
                for j in range(_D // _L):
                    s = pl.ds(j * _L, _L)
                    lv = lrb[i, s]
                    de = hrb[i, s] + lv - trb[i, s]
                    acc_e = acc_e + de * de
                    dc = crb[i, s] + lv - drb[i, s]
                    acc_c = acc_c + dc * dc
                se_s[pl.ds(i * 17, _L)] = acc_e
                sc_s[pl.ds(i * 17, _L)] = acc_c
                return carry

            lax.fori_loop(0, chunk, row_body, 0)

            def grp_body(q, carry):
                qbase = q * (17 * _L)
                tot_e = jnp.zeros((_L,), jnp.float32)
                tot_c = jnp.zeros((_L,), jnp.float32)
                for d in range(_L):
                    idx = iota17 + (qbase + d)
                    tot_e = tot_e + plsc.load_gather(se_s, [idx])
                    tot_c = tot_c + plsc.load_gather(sc_s, [idx])
                eo[pl.ds(off + q * _L, _L)] = tot_e
                co[pl.ds(off + q * _L, _L)] = tot_c
                return carry

            lax.fori_loop(0, chunk // _L, grp_body, 0)

        pend = fire(0, bufs[0])
        for g in range(nch):
            nxt = None
            if g + 1 < nch:
                nxt = fire(g + 1, bufs[(g + 1) % 2])
            for cp in pend:
                cp.wait()
            compute(g, bufs[g % 2])
            pend = nxt

        pltpu.sync_copy(eo, oe_hbm.at[pl.ds(base, bpw)])
        pltpu.sync_copy(co, oc_hbm.at[pl.ds(base, bpw)])

    return sck(h_i, l_i, t_i, ch_i, ct_i, obj, rel)


def _tc_finish(sum_e, sum_c):
    rows = _B // 128

    def body(se_ref, sc_ref, loss_ref, e_ref, ce_ref):
        e = jnp.sqrt(se_ref[...])
        ce = jnp.sqrt(sc_ref[...])
        e_ref[...] = e
        ce_ref[...] = ce
        loss_ref[...] = jnp.maximum(0.0, e - ce + _MARGIN)

    loss, e, ce = pl.pallas_call(
        body,
        out_shape=(
            jax.ShapeDtypeStruct((rows, 128), jnp.float32),
            jax.ShapeDtypeStruct((rows, 128), jnp.float32),
            jax.ShapeDtypeStruct((rows, 128), jnp.float32),
        ),
    )(sum_e.reshape(rows, 128), sum_c.reshape(rows, 128))
    return loss.reshape(_B), e.reshape(_B), ce.reshape(_B)


@jax.jit
def kernel(correct, corrupted, object_embedding, relationship_embedding):
    h_i = correct[:, 0]
    l_i = correct[:, 1]
    t_i = correct[:, 2]
    ch_i = corrupted[:, 0]
    ct_i = corrupted[:, 2]
    sum_e, sum_c = _sc_sumsq(h_i, l_i, t_i, ch_i, ct_i,
                             object_embedding, relationship_embedding)
    return _tc_finish(sum_e, sum_c)
